# baseline (speedup 1.0000x reference)
import functools

import jax
import jax.numpy as jnp
from jax import lax
from jax.experimental import pallas as pl
from jax.experimental.pallas import tpu as pltpu
from jax.experimental.pallas import tpu_sc as plsc

_NC = 2
_NS = 16
_NW = _NC * _NS
_LANES = 16
_TV = 1024


_RPS = 16


def _prep_body(v_hbm, u_hbm, g_ref, vs, us, vsem, usem):
    i = pl.program_id(0)
    voc = v_hbm.shape[0]

    @pl.when(i == 0)
    def _():
        cp1 = pltpu.make_async_copy(v_hbm, vs.at[pl.ds(0, voc)], vsem)
        cp2 = pltpu.make_async_copy(u_hbm, us, usem)
        cp1.start()
        cp2.start()
        cp1.wait()
        cp2.wait()

    rows = lax.dot_general(
        vs[pl.ds(i * _RPS, _RPS), :], us[...], (((1,), (1,)), ((), ())),
        preferred_element_type=jnp.float32,
        precision=lax.Precision.HIGHEST)
    padded = jnp.pad(rows, ((0, 0), (0, _TV - rows.shape[1])))
    g_ref[...] = padded.reshape(_RPS * _TV)


@functools.lru_cache(maxsize=None)
def _make_sc_gather(B, K1):
    bcols = B // _NW
    bpw = bcols * K1
    mesh = plsc.VectorSubcoreMesh(core_axis_name="c", subcore_axis_name="s")

    @functools.partial(
        pl.kernel, mesh=mesh,
        out_type=jax.ShapeDtypeStruct((_NW, 2 * _LANES), jnp.float32),
        scratch_types=[
            pltpu.VMEM((bcols,), jnp.int32),
            pltpu.VMEM((bpw,), jnp.int32),
            pltpu.VMEM((bpw,), jnp.int32),
            pltpu.VMEM((bpw,), jnp.float32),
            pltpu.VMEM((2 * _LANES,), jnp.float32),
            pltpu.SemaphoreType.DMA,
            pltpu.SemaphoreType.DMA,
        ])
    def sc_fn(g_hbm, cen_hbm, cnt_hbm, out_hbm,
              cen_v, cn_v, idx_v, val_v, st_v, isem, gsem):
        wid = lax.axis_index("s") * _NC + lax.axis_index("c")
        base = wid * bcols
        cps = [pltpu.async_copy(cen_hbm.at[pl.ds(base, bcols)], cen_v, isem)]
        cps += [
            pltpu.async_copy(cnt_hbm.at[k, pl.ds(base, bcols)],
                             cn_v.at[pl.ds(k * bcols, bcols)], isem)
            for k in range(K1)
        ]
        for cp in cps:
            cp.wait()

        for k in range(K1):
            for i in range(bcols // _LANES):
                o = k * bcols + i * _LANES
                c16 = cen_v[pl.ds(i * _LANES, _LANES)]
                n16 = cn_v[pl.ds(o, _LANES)]
                idx_v[pl.ds(o, _LANES)] = c16 * _TV + n16

        pltpu.async_copy(g_hbm.at[idx_v], val_v, gsem).wait()

        acc = jnp.zeros((_LANES,), jnp.float32)
        acc0 = jnp.zeros((_LANES,), jnp.float32)
        for i in range(bpw // _LANES):
            x = val_v[pl.ds(i * _LANES, _LANES)]
            acc = acc + jnp.exp(x)
            if i < bcols // _LANES:
                acc0 = acc0 + x
        st_v[pl.ds(0, _LANES)] = acc
        st_v[pl.ds(_LANES, _LANES)] = acc0
        pltpu.sync_copy(st_v, out_hbm.at[wid])

    return sc_fn


@functools.lru_cache(maxsize=None)
def _make_finalize(B):
    def _fin_body(p_ref, out_ref):
        s = jnp.sum(p_ref[:, 0:_LANES])
        l0 = jnp.sum(p_ref[:, _LANES:2 * _LANES])
        out_ref[...] = jnp.reshape(float(B) * jnp.log(s) - l0, (1, 1))

    return pl.pallas_call(
        _fin_body,
        out_shape=jax.ShapeDtypeStruct((1, 1), jnp.float32))


def kernel(V, U, centers, contexts_negs):
    voc, d = V.shape
    B = centers.shape[0]
    K1 = contexts_negs.shape[1]
    G_flat = pl.pallas_call(
        _prep_body,
        grid=(_TV // _RPS,),
        in_specs=[pl.BlockSpec(memory_space=pltpu.MemorySpace.HBM),
                  pl.BlockSpec(memory_space=pltpu.MemorySpace.HBM)],
        out_specs=pl.BlockSpec((_RPS * _TV,), lambda i: (i,)),
        out_shape=jax.ShapeDtypeStruct((_TV * _TV,), jnp.float32),
        scratch_shapes=[pltpu.VMEM((_TV, d), jnp.float32),
                        pltpu.VMEM((voc, d), jnp.float32),
                        pltpu.SemaphoreType.DMA,
                        pltpu.SemaphoreType.DMA],
    )(V, U)
    esum_lsum = _make_sc_gather(B, K1)(G_flat, centers, contexts_negs.T)
    loss = _make_finalize(B)(esum_lsum)
    return loss[0, 0]


# speedup vs baseline: 1.8080x; 1.8080x over previous
import functools

import jax
import jax.numpy as jnp
from jax import lax
from jax.experimental import pallas as pl
from jax.experimental.pallas import tpu as pltpu
from jax.experimental.pallas import tpu_sc as plsc

_NC = 2
_NS = 16
_NW = _NC * _NS
_LANES = 16
_TV = 1024


def _prep_body(v_ref, u_ref, cen_ref, cnt_ref, g_ref, idx_ref):
    voc = v_ref.shape[0]
    g_ref[0:voc, 0:voc] = lax.dot_general(
        v_ref[...], u_ref[...], (((1,), (1,)), ((), ())),
        preferred_element_type=jnp.float32,
        precision=lax.Precision.HIGHEST)
    idx_ref[...] = cen_ref[...] * _TV + cnt_ref[...]


@functools.lru_cache(maxsize=None)
def _make_sc_gather(B, K1):
    bcols = B // _NW
    bpw = bcols * K1
    mesh = plsc.VectorSubcoreMesh(core_axis_name="c", subcore_axis_name="s")

    @functools.partial(
        pl.kernel, mesh=mesh,
        out_type=jax.ShapeDtypeStruct((_NW, 2 * _LANES), jnp.float32),
        scratch_types=[
            pltpu.VMEM((bpw,), jnp.int32),
            pltpu.VMEM((bpw,), jnp.float32),
            pltpu.VMEM((2 * _LANES,), jnp.float32),
            pltpu.SemaphoreType.DMA,
            pltpu.SemaphoreType.DMA,
        ])
    def sc_fn(g_hbm, idx_hbm, out_hbm, idx_v, val_v, st_v, isem, gsem):
        wid = lax.axis_index("s") * _NC + lax.axis_index("c")
        base = wid * bcols
        idx_cps = [
            pltpu.async_copy(idx_hbm.at[k, pl.ds(base, bcols)],
                             idx_v.at[pl.ds(k * bcols, bcols)], isem)
            for k in range(K1)
        ]
        for cp in idx_cps:
            cp.wait()

        pltpu.async_copy(g_hbm.at[idx_v], val_v, gsem).wait()

        acc = jnp.zeros((_LANES,), jnp.float32)
        acc0 = jnp.zeros((_LANES,), jnp.float32)
        for i in range(bpw // _LANES):
            x = val_v[pl.ds(i * _LANES, _LANES)]
            acc = acc + jnp.exp(x)
            if i < bcols // _LANES:
                acc0 = acc0 + x
        st_v[pl.ds(0, _LANES)] = acc
        st_v[pl.ds(_LANES, _LANES)] = acc0
        pltpu.sync_copy(st_v, out_hbm.at[wid])

    return sc_fn


@functools.lru_cache(maxsize=None)
def _make_finalize(B):
    def _fin_body(p_ref, out_ref):
        s = jnp.sum(p_ref[:, 0:_LANES])
        l0 = jnp.sum(p_ref[:, _LANES:2 * _LANES])
        out_ref[...] = jnp.reshape(float(B) * jnp.log(s) - l0, (1, 1))

    return pl.pallas_call(
        _fin_body,
        out_shape=jax.ShapeDtypeStruct((1, 1), jnp.float32))


def kernel(V, U, centers, contexts_negs):
    voc, d = V.shape
    B = centers.shape[0]
    K1 = contexts_negs.shape[1]
    G, idx = pl.pallas_call(
        _prep_body,
        out_shape=(jax.ShapeDtypeStruct((_TV, _TV), jnp.float32),
                   jax.ShapeDtypeStruct((K1, B), jnp.int32)),
    )(V, U, centers[None, :], contexts_negs.T)
    esum_lsum = _make_sc_gather(B, K1)(G.reshape(_TV * _TV), idx)
    loss = _make_finalize(B)(esum_lsum)
    return loss[0, 0]
